# baseline (speedup 1.0000x reference)
import jax
import jax.numpy as jnp
from jax import lax
from jax.experimental import pallas as pl
from jax.experimental.pallas import tpu as pltpu

_N_EMB = 1024
_DIM = 64
_BATCH = 16
_TOK = 1024
_SLABS = 8
_GRID = _BATCH // _SLABS
_TOKENS = _BATCH * _TOK
_LANES = 128
_NCHUNK = _N_EMB // _LANES


def _vq_body(xt_ref, et_ref, qt_ref, idx_ref, losssum_ref):
    i = pl.program_id(0)
    et = et_ref[...]
    e2 = jnp.sum(et * et, axis=0, keepdims=True)
    e2c = e2.reshape(_N_EMB, 1)
    et_bf = et.astype(jnp.bfloat16)
    msum = jnp.float32(0.0)
    for b in range(_SLABS):
        xt = xt_ref[b]
        x2 = jnp.sum(xt * xt, axis=0, keepdims=True)
        xe = lax.dot_general(et, xt * (-2.0), (((0,), (0,)), ((), ())),
                             preferred_element_type=jnp.float32)
        d = (x2 + e2c) + xe
        runm = d[0:_LANES, :]
        runc = jnp.zeros((_LANES, _TOK), jnp.int32)
        for c in range(1, _NCHUNK):
            dc = d[c * _LANES:(c + 1) * _LANES, :]
            lt = dc < runm
            runm = jnp.where(lt, dc, runm)
            runc = jnp.where(lt, c, runc)
        m = jnp.min(runm, axis=0, keepdims=True)
        row = lax.broadcasted_iota(jnp.int32, (_LANES, _TOK), 0)
        cand = jnp.where(runm == m, runc * _LANES + row, _N_EMB)
        idx = jnp.min(cand, axis=0)
        idx_ref[b * (_TOK // _LANES):(b + 1) * (_TOK // _LANES), :] = (
            idx.reshape(_TOK // _LANES, _LANES))

        crow = lax.broadcasted_iota(jnp.int32, (_N_EMB, _TOK), 0)
        oht = (crow == idx[None, :]).astype(jnp.bfloat16)
        qt_ref[b] = lax.dot_general(
            et_bf, oht, (((1,), (0,)), ((), ())),
            preferred_element_type=jnp.float32)
        msum += jnp.sum(m)

    @pl.when(i == 0)
    def _init():
        losssum_ref[0, 0] = 0.0

    losssum_ref[0, 0] += msum

    @pl.when(i == pl.num_programs(0) - 1)
    def _finalize():
        losssum_ref[0, 0] *= 1.0 / float(_TOKENS * _DIM)


@jax.jit
def kernel(inputs, embeddings):
    xt = inputs.transpose(0, 2, 1)
    et = embeddings.T
    qt, idx128, losssum = pl.pallas_call(
        _vq_body,
        grid=(_GRID,),
        in_specs=[
            pl.BlockSpec((_SLABS, _DIM, _TOK), lambda i: (i, 0, 0)),
            pl.BlockSpec((_DIM, _N_EMB), lambda i: (0, 0)),
        ],
        out_specs=[
            pl.BlockSpec((_SLABS, _DIM, _TOK), lambda i: (i, 0, 0)),
            pl.BlockSpec((_SLABS * _TOK // _LANES, _LANES), lambda i: (i, 0)),
            pl.BlockSpec(memory_space=pltpu.SMEM),
        ],
        out_shape=[
            jax.ShapeDtypeStruct((_BATCH, _DIM, _TOK), jnp.float32),
            jax.ShapeDtypeStruct((_TOKENS // _LANES, _LANES), jnp.int32),
            jax.ShapeDtypeStruct((1, 1), jnp.float32),
        ],
    )(xt, et)
    q = qt.transpose(0, 2, 1)
    return q, losssum[0, 0], idx128.reshape(_TOKENS)[:, None]


# speedup vs baseline: 1.0088x; 1.0088x over previous
import jax
import jax.numpy as jnp
from jax import lax
from jax.experimental import pallas as pl
from jax.experimental.pallas import tpu as pltpu

_N_EMB = 1024
_DIM = 64
_BATCH = 16
_TOK = 1024
_SLABS = 4
_GRID = _BATCH // _SLABS
_TOKENS = _BATCH * _TOK
_LANES = 128
_NCHUNK = _N_EMB // _LANES


def _vq_body(xt_ref, et_ref, qt_ref, idx_ref, losssum_ref):
    i = pl.program_id(0)
    et = et_ref[...]
    e2 = jnp.sum(et * et, axis=0, keepdims=True)
    e2c = e2.reshape(_N_EMB, 1)
    et_bf = et.astype(jnp.bfloat16)
    msum = jnp.float32(0.0)
    for b in range(_SLABS):
        xt = xt_ref[b]
        x2 = jnp.sum(xt * xt, axis=0, keepdims=True)
        xe = lax.dot_general(et, xt * (-2.0), (((0,), (0,)), ((), ())),
                             preferred_element_type=jnp.float32)
        d = (x2 + e2c) + xe
        runm = d[0:_LANES, :]
        runc = jnp.zeros((_LANES, _TOK), jnp.int32)
        for c in range(1, _NCHUNK):
            dc = d[c * _LANES:(c + 1) * _LANES, :]
            lt = dc < runm
            runm = jnp.where(lt, dc, runm)
            runc = jnp.where(lt, c, runc)
        m = jnp.min(runm, axis=0, keepdims=True)
        row = lax.broadcasted_iota(jnp.int32, (_LANES, _TOK), 0)
        cand = jnp.where(runm == m, runc * _LANES + row, _N_EMB)
        idx = jnp.min(cand, axis=0)
        idx_ref[b * (_TOK // _LANES):(b + 1) * (_TOK // _LANES), :] = (
            idx.reshape(_TOK // _LANES, _LANES))

        crow = lax.broadcasted_iota(jnp.int32, (_N_EMB, _TOK), 0)
        oht = (crow == idx[None, :]).astype(jnp.bfloat16)
        qt_ref[b] = lax.dot_general(
            et_bf, oht, (((1,), (0,)), ((), ())),
            preferred_element_type=jnp.float32)
        msum += jnp.sum(m)

    @pl.when(i == 0)
    def _init():
        losssum_ref[0, 0] = 0.0

    losssum_ref[0, 0] += msum

    @pl.when(i == pl.num_programs(0) - 1)
    def _finalize():
        losssum_ref[0, 0] *= 1.0 / float(_TOKENS * _DIM)


@jax.jit
def kernel(inputs, embeddings):
    xt = inputs.transpose(0, 2, 1)
    et = embeddings.T
    qt, idx128, losssum = pl.pallas_call(
        _vq_body,
        grid=(_GRID,),
        in_specs=[
            pl.BlockSpec((_SLABS, _DIM, _TOK), lambda i: (i, 0, 0)),
            pl.BlockSpec((_DIM, _N_EMB), lambda i: (0, 0)),
        ],
        out_specs=[
            pl.BlockSpec((_SLABS, _DIM, _TOK), lambda i: (i, 0, 0)),
            pl.BlockSpec((_SLABS * _TOK // _LANES, _LANES), lambda i: (i, 0)),
            pl.BlockSpec(memory_space=pltpu.SMEM),
        ],
        out_shape=[
            jax.ShapeDtypeStruct((_BATCH, _DIM, _TOK), jnp.float32),
            jax.ShapeDtypeStruct((_TOKENS // _LANES, _LANES), jnp.int32),
            jax.ShapeDtypeStruct((1, 1), jnp.float32),
        ],
    )(xt, et)
    q = qt.transpose(0, 2, 1)
    return q, losssum[0, 0], idx128.reshape(_TOKENS)[:, None]
